# Initial kernel scaffold; baseline (speedup 1.0000x reference)
import jax
import jax.numpy as jnp
from jax.experimental import pallas as pl


def kernel(x, edge_index, Wl1, Wr1, att1, b1, ln1_g, ln1_b, Wsk1, bsk1, Wl2, Wr2, att2, b2, ln2_g, ln2_b, Wl3, Wr3, att3, b3, ln3_g, ln3_b, Wsk3, bsk3):
    raise NotImplementedError("write your pallas kernel here")


# speedup vs baseline: 15.5281x; 15.5281x over previous
import functools

import jax
import jax.numpy as jnp
from jax import lax
from jax.experimental import pallas as pl
from jax.experimental.pallas import tpu as pltpu
from jax.experimental.pallas import tpu_sc as plsc

N_NODES = 50000
N_EDGES = 800000
ET = N_EDGES + N_NODES
N_PAD = 50176
ROWS_PT = N_PAD // 16
CHUNK = 128
ET_PAD = 851968
EPT32 = ET_PAD // 32
EPT16 = ET_PAD // 16


def _mesh():
    return plsc.VectorSubcoreMesh(core_axis_name="c", subcore_axis_name="s",
                                  num_cores=2, num_subcores=16)


def _make_alpha_kernel(H, D):
    NK = D // 16

    @functools.partial(
        pl.kernel,
        out_type=jax.ShapeDtypeStruct((H, ET_PAD), jnp.float32),
        mesh=_mesh(),
        compiler_params=pltpu.CompilerParams(needs_layout_passes=False, use_tc_tiling_on_sc=False),
        scratch_types=[
            pltpu.VMEM((CHUNK,), jnp.int32),
            pltpu.VMEM((CHUNK,), jnp.int32),
            pltpu.VMEM((CHUNK, D), jnp.float32),
            pltpu.VMEM((CHUNK, D), jnp.float32),
            pltpu.VMEM((D,), jnp.float32),
            pltpu.VMEM((H, CHUNK), jnp.float32),
            pltpu.VMEM((256,), jnp.float32),
            pltpu.SemaphoreType.DMA,
            pltpu.SemaphoreType.DMA,
        ],
    )
    def alpha_kernel(xl, xr, src, dst, att, alpha_out,
                     src_v, dst_v, xj, xi, att_v, abuf, tbuf, sem1, sem2):
        wid = lax.axis_index("s") * 2 + lax.axis_index("c")
        base0 = wid * EPT32
        pltpu.sync_copy(att, att_v)
        att_vecs = [att_v[pl.ds(16 * k, 16)] for k in range(NK)]
        lanes = lax.iota(jnp.int32, 16)

        def chunk_body(ci, carry):
            base = base0 + ci * CHUNK
            pltpu.sync_copy(src.at[pl.ds(base, CHUNK)], src_v)
            pltpu.sync_copy(dst.at[pl.ds(base, CHUNK)], dst_v)
            cp1 = pltpu.async_copy(xl.at[src_v], xj, sem1)
            cp2 = pltpu.async_copy(xr.at[dst_v], xi, sem2)
            cp1.wait()
            cp2.wait()

            def grp_body(g, c2):
                e0 = g * 16
                for h in range(H):
                    for e in range(16):
                        acc = None
                        for k in range(64 // 16):
                            c = h * 64 + 16 * k
                            v = xj[e0 + e, pl.ds(c, 16)] + xi[e0 + e, pl.ds(c, 16)]
                            v = jnp.maximum(v, 0.2 * v)
                            t = v * att_vecs[(h * 64 + 16 * k) // 16]
                            acc = t if acc is None else acc + t
                        tbuf[pl.ds(e * 16, 16)] = acc
                    tot = None
                    for k in range(16):
                        col = plsc.load_gather(tbuf, [lanes * 16 + k])
                        tot = col if tot is None else tot + col
                    abuf[h, pl.ds(e0, 16)] = tot
                return c2

            lax.fori_loop(0, CHUNK // 16, grp_body, 0)
            for h in range(H):
                pltpu.sync_copy(abuf.at[h], alpha_out.at[h, pl.ds(base, CHUNK)])
            return carry

        lax.fori_loop(0, EPT32 // CHUNK, chunk_body, 0)

    return alpha_kernel


def _make_denom_kernel(H):
    @functools.partial(
        pl.kernel,
        out_type=jax.ShapeDtypeStruct((2, H, N_PAD), jnp.float32),
        mesh=_mesh(),
        compiler_params=pltpu.CompilerParams(needs_layout_passes=False, use_tc_tiling_on_sc=False),
        scratch_types=[
            pltpu.VMEM((CHUNK,), jnp.int32),
            pltpu.VMEM((CHUNK,), jnp.float32),
            pltpu.VMEM((ROWS_PT,), jnp.float32),
        ] + [pltpu.VMEM_SHARED((N_PAD,), jnp.float32) for _ in range(H)],
    )
    def denom_kernel(P, dst, out, dst_v, p_v, zbuf, *shared):
        cid = lax.axis_index("c")
        sid = lax.axis_index("s")
        wid = sid * 2 + cid
        zero16 = jnp.zeros((16,), jnp.float32)

        def zb(i, c):
            zbuf[pl.ds(i * 16, 16)] = zero16
            return c

        lax.fori_loop(0, ROWS_PT // 16, zb, 0)
        for h in range(H):
            pltpu.sync_copy(zbuf, shared[h].at[pl.ds(sid * ROWS_PT, ROWS_PT)])
        plsc.subcore_barrier()

        def chunk_body(ci, c):
            base = wid * EPT32 + ci * CHUNK
            pltpu.sync_copy(dst.at[pl.ds(base, CHUNK)], dst_v)
            for h in range(H):
                pltpu.sync_copy(P.at[h, pl.ds(base, CHUNK)], p_v)
                pltpu.sync_copy(p_v, shared[h].at[dst_v], add=True)
            return c

        lax.fori_loop(0, EPT32 // CHUNK, chunk_body, 0)
        plsc.subcore_barrier()
        for h in range(H):
            pltpu.sync_copy(shared[h].at[pl.ds(sid * ROWS_PT, ROWS_PT)],
                            out.at[cid, h, pl.ds(sid * ROWS_PT, ROWS_PT)])

    return denom_kernel


def _make_weight_kernel(H):
    HN = H * N_PAD
    STR = HN // 16

    @functools.partial(
        pl.kernel,
        out_type=jax.ShapeDtypeStruct((H, ET_PAD), jnp.float32),
        mesh=_mesh(),
        compiler_params=pltpu.CompilerParams(needs_layout_passes=False, use_tc_tiling_on_sc=False),
        scratch_types=[
            pltpu.VMEM((CHUNK,), jnp.int32),
            pltpu.VMEM((CHUNK,), jnp.int32),
            pltpu.VMEM((CHUNK,), jnp.float32),
            pltpu.VMEM((CHUNK,), jnp.float32),
            pltpu.VMEM((CHUNK,), jnp.float32),
            pltpu.VMEM_SHARED((HN,), jnp.float32),
            pltpu.SemaphoreType.DMA,
        ],
    )
    def weight_kernel(P, dst, dfull, W, dst_v, idx_v, p_v, d_v, w_v, dsp, sem):
        cid = lax.axis_index("c")
        sid = lax.axis_index("s")
        wid = sid * 2 + cid
        pltpu.sync_copy(dfull.at[pl.ds(sid * STR, STR)],
                        dsp.at[pl.ds(sid * STR, STR)])
        plsc.subcore_barrier()

        def chunk_body(ci, c):
            base = wid * EPT32 + ci * CHUNK
            pltpu.sync_copy(dst.at[pl.ds(base, CHUNK)], dst_v)
            for h in range(H):
                pltpu.sync_copy(P.at[h, pl.ds(base, CHUNK)], p_v)
                if h == 0:
                    idx_ref = dst_v
                else:
                    def mk(i, c2):
                        idx_v[pl.ds(i * 16, 16)] = (
                            dst_v[pl.ds(i * 16, 16)] + h * N_PAD)
                        return c2
                    lax.fori_loop(0, CHUNK // 16, mk, 0)
                    idx_ref = idx_v
                pltpu.async_copy(dsp.at[idx_ref], d_v, sem).wait()

                def dv(i, c2):
                    w_v[pl.ds(i * 16, 16)] = (
                        p_v[pl.ds(i * 16, 16)] / d_v[pl.ds(i * 16, 16)])
                    return c2

                lax.fori_loop(0, CHUNK // 16, dv, 0)
                pltpu.sync_copy(w_v, W.at[h, pl.ds(base, CHUNK)])
            return c

        lax.fori_loop(0, EPT32 // CHUNK, chunk_body, 0)

    return weight_kernel


def _make_agg_kernel(H, D):
    NS = D // 32
    PHASES = NS // 2
    ZR = 196

    @functools.partial(
        pl.kernel,
        out_type=jax.ShapeDtypeStruct((NS, N_PAD, 32), jnp.float32),
        mesh=_mesh(),
        compiler_params=pltpu.CompilerParams(needs_layout_passes=False, use_tc_tiling_on_sc=False),
        scratch_types=[
            pltpu.VMEM((CHUNK,), jnp.int32),
            pltpu.VMEM((CHUNK,), jnp.int32),
            pltpu.VMEM((CHUNK,), jnp.int32),
            pltpu.VMEM((CHUNK,), jnp.float32),
            pltpu.VMEM((CHUNK, 32), jnp.float32),
            pltpu.VMEM((CHUNK, 32), jnp.float32),
            pltpu.VMEM((ZR, 32), jnp.float32),
            pltpu.VMEM_SHARED((N_PAD, 32), jnp.float32),
            pltpu.SemaphoreType.DMA,
        ],
    )
    def agg_kernel(xls, srcr, dstr, Wf, agg,
                   src_v, dst_v, idx_v, w_v, rows, msg, zrows, acc, sem):
        cid = lax.axis_index("c")
        sid = lax.axis_index("s")
        zero16 = jnp.zeros((16,), jnp.float32)
        zidx = jnp.zeros((16,), jnp.int32)

        def zz(i, c):
            for q in range(2):
                zrows[i, pl.ds(q * 16, 16)] = zero16
            return c

        lax.fori_loop(0, ZR, zz, 0)

        for phase in range(PHASES):
            s = cid * PHASES + phase
            hs = (s * H) // NS
            woff = hs * ET_PAD
            soff = s * N_PAD

            def zr(j, c):
                r0 = sid * ROWS_PT + j * ZR
                pltpu.sync_copy(zrows, acc.at[pl.ds(r0, ZR)])
                return c

            lax.fori_loop(0, ROWS_PT // ZR, zr, 0)
            plsc.subcore_barrier()

            def chunk_body(ci, c):
                base = sid * EPT16 + ci * CHUNK
                pltpu.sync_copy(srcr.at[pl.ds(base, CHUNK)], src_v)
                pltpu.sync_copy(dstr.at[pl.ds(base, CHUNK)], dst_v)
                pltpu.sync_copy(Wf.at[pl.ds(woff + base, CHUNK)], w_v)

                def mk(i, c2):
                    idx_v[pl.ds(i * 16, 16)] = src_v[pl.ds(i * 16, 16)] + soff
                    return c2

                lax.fori_loop(0, CHUNK // 16, mk, 0)
                pltpu.async_copy(xls.at[idx_v], rows, sem).wait()

                def pe(e, c2):
                    w16 = plsc.load_gather(w_v, [zidx + e])
                    for q in range(2):
                        msg[e, pl.ds(q * 16, 16)] = (
                            rows[e, pl.ds(q * 16, 16)] * w16)
                    return c2

                lax.fori_loop(0, CHUNK, pe, 0)
                pltpu.sync_copy(msg, acc.at[dst_v], add=True)
                return c

            lax.fori_loop(0, EPT16 // CHUNK, chunk_body, 0)
            plsc.subcore_barrier()

            def fl(j, c):
                r0 = sid * ROWS_PT + j * ZR
                pltpu.sync_copy(acc.at[pl.ds(r0, ZR)], agg.at[s, pl.ds(r0, ZR)])
                return c

            lax.fori_loop(0, ROWS_PT // ZR, fl, 0)
            if phase + 1 < PHASES:
                plsc.subcore_barrier()

    return agg_kernel


def _make_mm_kernel(K, splits):
    Dtot = sum(splits)
    NSxl = splits[0] // 32
    BN = 512
    grid = N_PAD // BN

    def body(x_ref, w_ref, *out_refs):
        y = jnp.dot(x_ref[...], w_ref[...], preferred_element_type=jnp.float32)
        off = 0
        for j, d in enumerate(splits):
            out_refs[j][...] = y[:, off:off + d]
            off += d
        for si in range(NSxl):
            out_refs[-1][si] = y[:, si * 32:(si + 1) * 32]

    outs = ([jax.ShapeDtypeStruct((N_PAD, d), jnp.float32) for d in splits]
            + [jax.ShapeDtypeStruct((NSxl, N_PAD, 32), jnp.float32)])
    out_specs = ([pl.BlockSpec((BN, d), lambda i: (i, 0)) for d in splits]
                 + [pl.BlockSpec((NSxl, BN, 32), lambda i: (0, i, 0))])
    return pl.pallas_call(
        body,
        grid=(grid,),
        in_specs=[pl.BlockSpec((BN, K), lambda i: (i, 0)),
                  pl.BlockSpec((K, Dtot), lambda i: (0, 0))],
        out_specs=out_specs,
        out_shape=outs,
    )


def _make_maxred_kernel(H):
    BC = 4096
    grid = ET_PAD // BC

    def body(a_ref, o_ref):
        i = pl.program_id(0)
        m = jnp.full((1, 1), jnp.max(a_ref[...]))

        @pl.when(i == 0)
        def _():
            o_ref[...] = m

        @pl.when(i > 0)
        def _():
            o_ref[...] = jnp.maximum(o_ref[...], m)

    return pl.pallas_call(
        body,
        grid=(grid,),
        in_specs=[pl.BlockSpec((H, BC), lambda i: (0, i))],
        out_specs=pl.BlockSpec((1, 1), lambda i: (0, 0)),
        out_shape=jax.ShapeDtypeStruct((1, 1), jnp.float32),
    )


def _make_expsub_kernel(H):
    BC = 4096
    grid = ET_PAD // BC

    def body(a_ref, c_ref, o_ref):
        o_ref[...] = jnp.exp(a_ref[...] - c_ref[0, 0])

    return pl.pallas_call(
        body,
        grid=(grid,),
        in_specs=[pl.BlockSpec((H, BC), lambda i: (0, i)),
                  pl.BlockSpec((1, 1), lambda i: (0, 0))],
        out_specs=pl.BlockSpec((H, BC), lambda i: (0, i)),
        out_shape=jax.ShapeDtypeStruct((H, ET_PAD), jnp.float32),
    )


def _make_combine_kernel(H):
    def body(p_ref, o_ref):
        o_ref[...] = p_ref[0] + p_ref[1] + 1e-16

    return pl.pallas_call(
        body,
        in_specs=[pl.BlockSpec((2, H, N_PAD), lambda: (0, 0, 0))],
        out_specs=pl.BlockSpec((H, N_PAD), lambda: (0, 0)),
        out_shape=jax.ShapeDtypeStruct((H, N_PAD), jnp.float32),
    )


def _make_post_kernel(D):
    NS = D // 32
    BN = 256
    grid = N_PAD // BN

    def body(a_ref, b_ref, g_ref, bl_ref, sk_ref, skb_ref, o_ref):
        a = a_ref[...]
        v = jnp.concatenate([a[s] for s in range(NS)], axis=-1) + b_ref[...]
        m = jnp.mean(v, axis=-1, keepdims=True)
        var = jnp.mean((v - m) ** 2, axis=-1, keepdims=True)
        vn = (v - m) * lax.rsqrt(var + 1e-5) * g_ref[...] + bl_ref[...]
        e = jnp.where(vn > 0, vn, jnp.exp(jnp.minimum(vn, 0.0)) - 1.0)
        o_ref[...] = e + sk_ref[...] + skb_ref[...]

    return pl.pallas_call(
        body,
        grid=(grid,),
        in_specs=[pl.BlockSpec((NS, BN, 32), lambda i: (0, i, 0)),
                  pl.BlockSpec((1, D), lambda i: (0, 0)),
                  pl.BlockSpec((1, D), lambda i: (0, 0)),
                  pl.BlockSpec((1, D), lambda i: (0, 0)),
                  pl.BlockSpec((BN, D), lambda i: (i, 0)),
                  pl.BlockSpec((1, D), lambda i: (0, 0))],
        out_specs=pl.BlockSpec((BN, D), lambda i: (i, 0)),
        out_shape=jax.ShapeDtypeStruct((N_PAD, D), jnp.float32),
    )


def _make_pool_kernel():
    BN = 512
    grid = N_PAD // BN

    def body(h_ref, o_ref):
        i = pl.program_id(0)
        rows = i * BN + lax.broadcasted_iota(jnp.int32, (BN, 64), 0)
        valid = rows < N_NODES
        hb = h_ref[...]
        s = jnp.sum(jnp.where(valid, hb, 0.0), axis=0)
        mx = jnp.max(jnp.where(valid, hb, -jnp.inf), axis=0)

        @pl.when(i == 0)
        def _():
            o_ref[0, :] = s
            o_ref[1, :] = mx

        @pl.when(i > 0)
        def _():
            o_ref[0, :] = o_ref[0, :] + s
            o_ref[1, :] = jnp.maximum(o_ref[1, :], mx)

        @pl.when(i == grid - 1)
        def _():
            o_ref[0, :] = o_ref[0, :] * (1.0 / N_NODES)

    return pl.pallas_call(
        body,
        grid=(grid,),
        in_specs=[pl.BlockSpec((BN, 64), lambda i: (i, 0))],
        out_specs=pl.BlockSpec((2, 64), lambda i: (0, 0)),
        out_shape=jax.ShapeDtypeStruct((2, 64), jnp.float32),
    )


_alpha_2 = _make_alpha_kernel(2, 128)
_alpha_1 = _make_alpha_kernel(1, 64)
_denom_2 = _make_denom_kernel(2)
_denom_1 = _make_denom_kernel(1)
_weight_2 = _make_weight_kernel(2)
_weight_1 = _make_weight_kernel(1)
_agg_2 = _make_agg_kernel(2, 128)
_agg_1 = _make_agg_kernel(1, 64)
_mm_1 = _make_mm_kernel(16, (128, 128, 128))
_mm_2 = _make_mm_kernel(128, (128, 128))
_mm_3 = _make_mm_kernel(128, (64, 64, 64))
_maxred_2 = _make_maxred_kernel(2)
_maxred_1 = _make_maxred_kernel(1)
_expsub_2 = _make_expsub_kernel(2)
_expsub_1 = _make_expsub_kernel(1)
_combine_2 = _make_combine_kernel(2)
_combine_1 = _make_combine_kernel(1)
_post_128 = _make_post_kernel(128)
_post_64 = _make_post_kernel(64)
_pool = _make_pool_kernel()


def _gat_layer(xl, xr, xls, src, dst, att, b, g, bl, sk, skb, H, D):
    NS = D // 32
    alpha_k = _alpha_2 if H == 2 else _alpha_1
    denom_k = _denom_2 if H == 2 else _denom_1
    weight_k = _weight_2 if H == 2 else _weight_1
    agg_k = _agg_2 if H == 2 else _agg_1
    maxred = _maxred_2 if H == 2 else _maxred_1
    expsub = _expsub_2 if H == 2 else _expsub_1
    combine = _combine_2 if H == 2 else _combine_1
    post = _post_128 if D == 128 else _post_64

    alpha = alpha_k(xl, xr, src, dst, att.reshape(-1))
    cmax = maxred(alpha)
    P = expsub(alpha, cmax)
    partials = denom_k(P, dst)
    dfull = combine(partials)
    W = weight_k(P, dst, dfull.reshape(-1))
    agg = agg_k(xls.reshape(NS * N_PAD, 32), src, dst, W.reshape(-1))
    return post(agg, b.reshape(1, D), g.reshape(1, D), bl.reshape(1, D),
                sk, skb.reshape(1, D))


def kernel(x, edge_index, Wl1, Wr1, att1, b1, ln1_g, ln1_b, Wsk1, bsk1,
           Wl2, Wr2, att2, b2, ln2_g, ln2_b,
           Wl3, Wr3, att3, b3, ln3_g, ln3_b, Wsk3, bsk3):
    f32 = jnp.float32
    loop = jnp.arange(N_NODES, dtype=jnp.int32)
    pad = jnp.full((ET_PAD - ET,), N_NODES, jnp.int32)
    src = jnp.concatenate([edge_index[0].astype(jnp.int32), loop, pad])
    dst = jnp.concatenate([edge_index[1].astype(jnp.int32), loop, pad])

    xp = jnp.zeros((N_PAD, 16), f32).at[:N_NODES, :14].set(x)
    Wt1 = jnp.zeros((16, 384), f32).at[:14].set(
        jnp.concatenate([Wl1.T, Wr1.T, Wsk1.T], axis=1))
    xl1, xr1, xsk1, xl1s = _mm_1(xp, Wt1)
    h1 = _gat_layer(xl1, xr1, xl1s, src, dst, att1, b1, ln1_g, ln1_b,
                    xsk1, bsk1, H=2, D=128)

    Wt2 = jnp.concatenate([Wl2.T, Wr2.T], axis=1)
    xl2, xr2, xl2s = _mm_2(h1, Wt2)
    h2 = _gat_layer(xl2, xr2, xl2s, src, dst, att2, b2, ln2_g, ln2_b,
                    h1, jnp.zeros((128,), f32), H=2, D=128)

    Wt3 = jnp.concatenate([Wl3.T, Wr3.T, Wsk3.T], axis=1)
    xl3, xr3, xsk3, xl3s = _mm_3(h2, Wt3)
    h3 = _gat_layer(xl3, xr3, xl3s, src, dst, att3, b3, ln3_g, ln3_b,
                    xsk3, bsk3, H=1, D=64)

    pooled = _pool(h3)
    return pooled.reshape(1, 128)


# speedup vs baseline: 19.8049x; 1.2754x over previous
import functools

import jax
import jax.numpy as jnp
from jax import lax
from jax.experimental import pallas as pl
from jax.experimental.pallas import tpu as pltpu
from jax.experimental.pallas import tpu_sc as plsc

N_NODES = 50000
N_EDGES = 800000
ET = N_EDGES + N_NODES
N_PAD = 50176
ROWS_PT = N_PAD // 16
CHUNK = 128
ET_PAD = 851968
EPT32 = ET_PAD // 32
EPT16 = ET_PAD // 16


def _mesh():
    return plsc.VectorSubcoreMesh(core_axis_name="c", subcore_axis_name="s",
                                  num_cores=2, num_subcores=16)


def _make_alpha_kernel(H, D):
    NK = D // 16
    BURST = 2
    BE = BURST * CHUNK

    @functools.partial(
        pl.kernel,
        out_type=jax.ShapeDtypeStruct((H, ET_PAD), jnp.float32),
        mesh=_mesh(),
        compiler_params=pltpu.CompilerParams(needs_layout_passes=False, use_tc_tiling_on_sc=False),
        scratch_types=[
            pltpu.VMEM((BURST, CHUNK), jnp.int32),
            pltpu.VMEM((BURST, CHUNK), jnp.int32),
            pltpu.VMEM((BE, D), jnp.float32),
            pltpu.VMEM((BE, D), jnp.float32),
            pltpu.VMEM((D,), jnp.float32),
            pltpu.VMEM((H, BE), jnp.float32),
            pltpu.VMEM((256,), jnp.float32),
            pltpu.SemaphoreType.DMA,
        ],
    )
    def alpha_kernel(xl, xr, src2, dst2, att, alpha_out,
                     src_v, dst_v, xj, xi, att_v, abuf, tbuf, semg):
        wid = lax.axis_index("s") * 2 + lax.axis_index("c")
        base0 = wid * EPT32
        row0_0 = wid * (EPT32 // CHUNK)
        pltpu.sync_copy(att, att_v)
        att_vecs = [att_v[pl.ds(16 * k, 16)] for k in range(NK)]
        lanes = lax.iota(jnp.int32, 16)

        def burst_body(t, carry):
            base = base0 + t * BE
            row0 = row0_0 + t * BURST
            pltpu.sync_copy(src2.at[pl.ds(row0, BURST)], src_v)
            pltpu.sync_copy(dst2.at[pl.ds(row0, BURST)], dst_v)
            cps = []
            for k in range(BURST):
                cps.append(pltpu.async_copy(
                    xl.at[src_v.at[k]], xj.at[pl.ds(k * CHUNK, CHUNK)], semg))
                cps.append(pltpu.async_copy(
                    xr.at[dst_v.at[k]], xi.at[pl.ds(k * CHUNK, CHUNK)], semg))
            for cp in cps:
                cp.wait()

            def grp_body(g, c2):
                e0 = g * 16
                for h in range(H):
                    for e in range(16):
                        acc = None
                        for k in range(64 // 16):
                            c = h * 64 + 16 * k
                            v = xj[e0 + e, pl.ds(c, 16)] + xi[e0 + e, pl.ds(c, 16)]
                            v = jnp.maximum(v, 0.2 * v)
                            t2 = v * att_vecs[(h * 64 + 16 * k) // 16]
                            acc = t2 if acc is None else acc + t2
                        tbuf[pl.ds(e * 16, 16)] = acc
                    tot = None
                    for k in range(16):
                        col = plsc.load_gather(tbuf, [lanes * 16 + k])
                        tot = col if tot is None else tot + col
                    abuf[h, pl.ds(e0, 16)] = tot
                return c2

            lax.fori_loop(0, BE // 16, grp_body, 0)
            for h in range(H):
                pltpu.sync_copy(abuf.at[h], alpha_out.at[h, pl.ds(base, BE)])
            return carry

        lax.fori_loop(0, EPT32 // BE, burst_body, 0)

    return alpha_kernel


def _make_agg_kernel(H, D):
    NS = D // 16
    PHASES = NS // 2
    BURST = 4
    BE = BURST * CHUNK
    ZR = 196

    @functools.partial(
        pl.kernel,
        out_type=[jax.ShapeDtypeStruct((NS, N_PAD, 16), jnp.float32),
                  jax.ShapeDtypeStruct((H, N_PAD), jnp.float32)],
        mesh=_mesh(),
        compiler_params=pltpu.CompilerParams(needs_layout_passes=False, use_tc_tiling_on_sc=False),
        scratch_types=[
            pltpu.VMEM((BE,), jnp.int32),
            pltpu.VMEM((BURST, CHUNK), jnp.int32),
            pltpu.VMEM((BURST, CHUNK), jnp.int32),
            pltpu.VMEM((BE,), jnp.float32),
            pltpu.VMEM((BE, 16), jnp.float32),
            pltpu.VMEM((BE, 16), jnp.float32),
            pltpu.VMEM((ZR, 16), jnp.float32),
            pltpu.VMEM((ROWS_PT,), jnp.float32),
            pltpu.VMEM_SHARED((N_PAD, 16), jnp.float32),
            pltpu.VMEM_SHARED((N_PAD,), jnp.float32),
            pltpu.SemaphoreType.DMA,
            pltpu.SemaphoreType.DMA,
        ],
    )
    def agg_kernel(xls, srcr, dst2, Pf, agg, denom,
                   src_v, dst_v, idx_v, p_v, rows, msg, zrows, zbuf,
                   acc, den_sp, semg, sems):
        cid = lax.axis_index("c")
        sid = lax.axis_index("s")
        zero16 = jnp.zeros((16,), jnp.float32)
        zidx = jnp.zeros((16,), jnp.int32)

        def zz(i, c):
            zrows[i, pl.ds(0, 16)] = zero16
            return c

        lax.fori_loop(0, ZR, zz, 0)

        def zb(i, c):
            zbuf[pl.ds(i * 16, 16)] = zero16
            return c

        lax.fori_loop(0, ROWS_PT // 16, zb, 0)
        pltpu.sync_copy(zbuf, den_sp.at[pl.ds(sid * ROWS_PT, ROWS_PT)])

        for phase in range(PHASES):
            s = cid * PHASES + phase
            hs = (s * H) // NS
            soff = s * N_PAD
            poff = hs * ET_PAD

            def zr(j, c):
                r0 = sid * ROWS_PT + j * ZR
                pltpu.sync_copy(zrows, acc.at[pl.ds(r0, ZR)])
                return c

            lax.fori_loop(0, ROWS_PT // ZR, zr, 0)
            plsc.subcore_barrier()

            def burst_body(t, c):
                base = sid * EPT16 + t * BE
                row0 = sid * (EPT16 // CHUNK) + t * BURST
                pltpu.sync_copy(srcr.at[pl.ds(base, BE)], src_v)
                pltpu.sync_copy(dst2.at[pl.ds(row0, BURST)], dst_v)
                pltpu.sync_copy(Pf.at[pl.ds(poff + base, BE)], p_v)

                def mk(i, c2):
                    k = i // 8
                    j = i % 8
                    idx_v[k, pl.ds(j * 16, 16)] = (
                        src_v[pl.ds(i * 16, 16)] + soff)
                    return c2

                lax.fori_loop(0, BE // 16, mk, 0)
                cps = [pltpu.async_copy(
                    xls.at[idx_v.at[k]], rows.at[pl.ds(k * CHUNK, CHUNK)], semg)
                    for k in range(BURST)]
                for cp in cps:
                    cp.wait()

                def pe(i, c2):
                    for u in range(4):
                        e = i * 4 + u
                        w16 = plsc.load_gather(p_v, [zidx + e])
                        msg[e, pl.ds(0, 16)] = rows[e, pl.ds(0, 16)] * w16
                    return c2

                lax.fori_loop(0, BE // 4, pe, 0)
                sc = [pltpu.async_copy(
                    msg.at[pl.ds(k * CHUNK, CHUNK)], acc.at[dst_v.at[k]],
                    sems, add=True) for k in range(BURST)]
                if phase == 0:
                    sc += [pltpu.async_copy(
                        p_v.at[pl.ds(k * CHUNK, CHUNK)], den_sp.at[dst_v.at[k]],
                        sems, add=True) for k in range(BURST)]
                for cp in sc:
                    cp.wait()
                return c

            lax.fori_loop(0, EPT16 // BE, burst_body, 0)
            plsc.subcore_barrier()

            def fl(j, c):
                r0 = sid * ROWS_PT + j * ZR
                pltpu.sync_copy(acc.at[pl.ds(r0, ZR)], agg.at[s, pl.ds(r0, ZR)])
                return c

            lax.fori_loop(0, ROWS_PT // ZR, fl, 0)
            if phase == 0:
                if H == 2:
                    pltpu.sync_copy(
                        den_sp.at[pl.ds(sid * ROWS_PT, ROWS_PT)],
                        denom.at[hs, pl.ds(sid * ROWS_PT, ROWS_PT)])
                else:
                    @pl.when(cid == 0)
                    def _():
                        pltpu.sync_copy(
                            den_sp.at[pl.ds(sid * ROWS_PT, ROWS_PT)],
                            denom.at[0, pl.ds(sid * ROWS_PT, ROWS_PT)])
            if phase + 1 < PHASES:
                plsc.subcore_barrier()

    return agg_kernel


def _make_mm_kernel(K, splits):
    Dtot = sum(splits)
    NSxl = splits[0] // 16
    BN = 512
    grid = N_PAD // BN

    def body(x_ref, w_ref, *out_refs):
        y = jnp.dot(x_ref[...], w_ref[...], preferred_element_type=jnp.float32)
        off = 0
        for j, d in enumerate(splits):
            out_refs[j][...] = y[:, off:off + d]
            off += d
        for si in range(NSxl):
            out_refs[-1][si] = y[:, si * 16:(si + 1) * 16]

    outs = ([jax.ShapeDtypeStruct((N_PAD, d), jnp.float32) for d in splits]
            + [jax.ShapeDtypeStruct((NSxl, N_PAD, 16), jnp.float32)])
    out_specs = ([pl.BlockSpec((BN, d), lambda i: (i, 0)) for d in splits]
                 + [pl.BlockSpec((NSxl, BN, 16), lambda i: (0, i, 0))])
    return pl.pallas_call(
        body,
        grid=(grid,),
        in_specs=[pl.BlockSpec((BN, K), lambda i: (i, 0)),
                  pl.BlockSpec((K, Dtot), lambda i: (0, 0))],
        out_specs=out_specs,
        out_shape=outs,
    )


def _make_maxred_kernel(H):
    BC = 4096
    grid = ET_PAD // BC

    def body(a_ref, o_ref):
        i = pl.program_id(0)
        m = jnp.full((1, 1), jnp.max(a_ref[...]))

        @pl.when(i == 0)
        def _():
            o_ref[...] = m

        @pl.when(i > 0)
        def _():
            o_ref[...] = jnp.maximum(o_ref[...], m)

    return pl.pallas_call(
        body,
        grid=(grid,),
        in_specs=[pl.BlockSpec((H, BC), lambda i: (0, i))],
        out_specs=pl.BlockSpec((1, 1), lambda i: (0, 0)),
        out_shape=jax.ShapeDtypeStruct((1, 1), jnp.float32),
    )


def _make_expsub_kernel(H):
    BC = 4096
    grid = ET_PAD // BC

    def body(a_ref, c_ref, o_ref):
        o_ref[...] = jnp.exp(a_ref[...] - c_ref[0, 0])

    return pl.pallas_call(
        body,
        grid=(grid,),
        in_specs=[pl.BlockSpec((H, BC), lambda i: (0, i)),
                  pl.BlockSpec((1, 1), lambda i: (0, 0))],
        out_specs=pl.BlockSpec((H, BC), lambda i: (0, i)),
        out_shape=jax.ShapeDtypeStruct((H, ET_PAD), jnp.float32),
    )


def _make_post_kernel(D, H):
    NS = D // 16
    BN = 256
    grid = N_PAD // BN

    def body(a_ref, d_ref, b_ref, g_ref, bl_ref, sk_ref, skb_ref, o_ref):
        a = a_ref[...]
        den = d_ref[...] + 1e-16
        parts = []
        for s in range(NS):
            hs = (s * H) // NS
            parts.append(a[s] / den[hs][:, None])
        v = jnp.concatenate(parts, axis=-1) + b_ref[...]
        m = jnp.mean(v, axis=-1, keepdims=True)
        var = jnp.mean((v - m) ** 2, axis=-1, keepdims=True)
        vn = (v - m) * lax.rsqrt(var + 1e-5) * g_ref[...] + bl_ref[...]
        e = jnp.where(vn > 0, vn, jnp.exp(jnp.minimum(vn, 0.0)) - 1.0)
        o_ref[...] = e + sk_ref[...] + skb_ref[...]

    return pl.pallas_call(
        body,
        grid=(grid,),
        in_specs=[pl.BlockSpec((NS, BN, 16), lambda i: (0, i, 0)),
                  pl.BlockSpec((H, BN), lambda i: (0, i)),
                  pl.BlockSpec((1, D), lambda i: (0, 0)),
                  pl.BlockSpec((1, D), lambda i: (0, 0)),
                  pl.BlockSpec((1, D), lambda i: (0, 0)),
                  pl.BlockSpec((BN, D), lambda i: (i, 0)),
                  pl.BlockSpec((1, D), lambda i: (0, 0))],
        out_specs=pl.BlockSpec((BN, D), lambda i: (i, 0)),
        out_shape=jax.ShapeDtypeStruct((N_PAD, D), jnp.float32),
    )


def _make_pool_kernel():
    BN = 512
    grid = N_PAD // BN

    def body(h_ref, o_ref):
        i = pl.program_id(0)
        rows = i * BN + lax.broadcasted_iota(jnp.int32, (BN, 64), 0)
        valid = rows < N_NODES
        hb = h_ref[...]
        s = jnp.sum(jnp.where(valid, hb, 0.0), axis=0)
        mx = jnp.max(jnp.where(valid, hb, -jnp.inf), axis=0)

        @pl.when(i == 0)
        def _():
            o_ref[0, :] = s
            o_ref[1, :] = mx

        @pl.when(i > 0)
        def _():
            o_ref[0, :] = o_ref[0, :] + s
            o_ref[1, :] = jnp.maximum(o_ref[1, :], mx)

        @pl.when(i == grid - 1)
        def _():
            o_ref[0, :] = o_ref[0, :] * (1.0 / N_NODES)

    return pl.pallas_call(
        body,
        grid=(grid,),
        in_specs=[pl.BlockSpec((BN, 64), lambda i: (i, 0))],
        out_specs=pl.BlockSpec((2, 64), lambda i: (0, 0)),
        out_shape=jax.ShapeDtypeStruct((2, 64), jnp.float32),
    )


_alpha_2 = _make_alpha_kernel(2, 128)
_alpha_1 = _make_alpha_kernel(1, 64)
_agg_2 = _make_agg_kernel(2, 128)
_agg_1 = _make_agg_kernel(1, 64)
_mm_1 = _make_mm_kernel(16, (128, 128, 128))
_mm_2 = _make_mm_kernel(128, (128, 128))
_mm_3 = _make_mm_kernel(128, (64, 64, 64))
_maxred_2 = _make_maxred_kernel(2)
_maxred_1 = _make_maxred_kernel(1)
_expsub_2 = _make_expsub_kernel(2)
_expsub_1 = _make_expsub_kernel(1)
_post_128 = _make_post_kernel(128, 2)
_post_64 = _make_post_kernel(64, 1)
_pool = _make_pool_kernel()


def _gat_layer(xl, xr, xls, src, src2, dst2, att, b, g, bl, sk, skb, H, D):
    NS = D // 16
    alpha_k = _alpha_2 if H == 2 else _alpha_1
    agg_k = _agg_2 if H == 2 else _agg_1
    maxred = _maxred_2 if H == 2 else _maxred_1
    expsub = _expsub_2 if H == 2 else _expsub_1
    post = _post_128 if D == 128 else _post_64

    alpha = alpha_k(xl, xr, src2, dst2, att.reshape(-1))
    cmax = maxred(alpha)
    P = expsub(alpha, cmax)
    agg, den = agg_k(xls.reshape(NS * N_PAD, 16), src, dst2, P.reshape(-1))
    return post(agg, den, b.reshape(1, D), g.reshape(1, D), bl.reshape(1, D),
                sk, skb.reshape(1, D))


def kernel(x, edge_index, Wl1, Wr1, att1, b1, ln1_g, ln1_b, Wsk1, bsk1,
           Wl2, Wr2, att2, b2, ln2_g, ln2_b,
           Wl3, Wr3, att3, b3, ln3_g, ln3_b, Wsk3, bsk3):
    f32 = jnp.float32
    loop = jnp.arange(N_NODES, dtype=jnp.int32)
    pad = jnp.full((ET_PAD - ET,), N_NODES, jnp.int32)
    src = jnp.concatenate([edge_index[0].astype(jnp.int32), loop, pad])
    dst = jnp.concatenate([edge_index[1].astype(jnp.int32), loop, pad])
    src2 = src.reshape(-1, CHUNK)
    dst2 = dst.reshape(-1, CHUNK)

    xp = jnp.zeros((N_PAD, 16), f32).at[:N_NODES, :14].set(x)
    Wt1 = jnp.zeros((16, 384), f32).at[:14].set(
        jnp.concatenate([Wl1.T, Wr1.T, Wsk1.T], axis=1))
    xl1, xr1, xsk1, xl1s = _mm_1(xp, Wt1)
    h1 = _gat_layer(xl1, xr1, xl1s, src, src2, dst2, att1, b1, ln1_g, ln1_b,
                    xsk1, bsk1, H=2, D=128)

    Wt2 = jnp.concatenate([Wl2.T, Wr2.T], axis=1)
    xl2, xr2, xl2s = _mm_2(h1, Wt2)
    h2 = _gat_layer(xl2, xr2, xl2s, src, src2, dst2, att2, b2, ln2_g, ln2_b,
                    h1, jnp.zeros((128,), f32), H=2, D=128)

    Wt3 = jnp.concatenate([Wl3.T, Wr3.T, Wsk3.T], axis=1)
    xl3, xr3, xsk3, xl3s = _mm_3(h2, Wt3)
    h3 = _gat_layer(xl3, xr3, xl3s, src, src2, dst2, att3, b3, ln3_g, ln3_b,
                    xsk3, bsk3, H=1, D=64)

    pooled = _pool(h3)
    return pooled.reshape(1, 128)


# speedup vs baseline: 20.9782x; 1.0592x over previous
import functools

import jax
import jax.numpy as jnp
from jax import lax
from jax.experimental import pallas as pl
from jax.experimental.pallas import tpu as pltpu
from jax.experimental.pallas import tpu_sc as plsc

N_NODES = 50000
N_EDGES = 800000
ET = N_EDGES + N_NODES
N_PAD = 50176
ROWS_PT = N_PAD // 16
CHUNK = 128
ET_PAD = 851968
EPT32 = ET_PAD // 32
EPT16 = ET_PAD // 16


def _mesh():
    return plsc.VectorSubcoreMesh(core_axis_name="c", subcore_axis_name="s",
                                  num_cores=2, num_subcores=16)


def _make_alpha_kernel(H, D):
    NK = D // 16
    BURST = 2
    BE = BURST * CHUNK

    @functools.partial(
        pl.kernel,
        out_type=[jax.ShapeDtypeStruct((H, ET_PAD), jnp.float32),
                  jax.ShapeDtypeStruct((2, H, N_PAD), jnp.float32)],
        mesh=_mesh(),
        compiler_params=pltpu.CompilerParams(needs_layout_passes=False, use_tc_tiling_on_sc=False),
        scratch_types=[
            pltpu.VMEM((BURST, CHUNK), jnp.int32),
            pltpu.VMEM((BURST, CHUNK), jnp.int32),
            pltpu.VMEM((BE, D), jnp.float32),
            pltpu.VMEM((BE, D), jnp.float32),
            pltpu.VMEM((D,), jnp.float32),
            pltpu.VMEM((H, BE), jnp.float32),
            pltpu.VMEM((256,), jnp.float32),
            pltpu.VMEM((ROWS_PT,), jnp.float32),
        ] + [pltpu.VMEM_SHARED((N_PAD,), jnp.float32) for _ in range(H)] + [
            pltpu.SemaphoreType.DMA,
            pltpu.SemaphoreType.DMA,
        ],
    )
    def alpha_kernel(xl, xr, src2, dst2, att, alpha_out, den_part,
                     src_v, dst_v, xj, xi, att_v, abuf, tbuf, zbuf, *rest):
        den_sh = rest[:H]
        semg = rest[H]
        semd = rest[H + 1]
        cid = lax.axis_index("c")
        sid = lax.axis_index("s")
        wid = sid * 2 + cid
        base0 = wid * EPT32
        row0_0 = wid * (EPT32 // CHUNK)
        pltpu.sync_copy(att, att_v)
        att_vecs = [att_v[pl.ds(16 * k, 16)] for k in range(NK)]
        lanes = lax.iota(jnp.int32, 16)
        zero16 = jnp.zeros((16,), jnp.float32)

        def zb(i, c):
            zbuf[pl.ds(i * 16, 16)] = zero16
            return c

        lax.fori_loop(0, ROWS_PT // 16, zb, 0)
        for h in range(H):
            pltpu.sync_copy(zbuf, den_sh[h].at[pl.ds(sid * ROWS_PT, ROWS_PT)])
        plsc.subcore_barrier()

        def burst_body(t, carry):
            base = base0 + t * BE
            row0 = row0_0 + t * BURST
            pltpu.sync_copy(src2.at[pl.ds(row0, BURST)], src_v)
            pltpu.sync_copy(dst2.at[pl.ds(row0, BURST)], dst_v)
            cps = []
            for k in range(BURST):
                cps.append(pltpu.async_copy(
                    xl.at[src_v.at[k]], xj.at[pl.ds(k * CHUNK, CHUNK)], semg))
                cps.append(pltpu.async_copy(
                    xr.at[dst_v.at[k]], xi.at[pl.ds(k * CHUNK, CHUNK)], semg))
            for cp in cps:
                cp.wait()

            def grp_body(g, c2):
                e0 = g * 16
                for h in range(H):
                    for e in range(16):
                        acc = None
                        for k in range(64 // 16):
                            c = h * 64 + 16 * k
                            v = xj[e0 + e, pl.ds(c, 16)] + xi[e0 + e, pl.ds(c, 16)]
                            v = jnp.maximum(v, 0.2 * v)
                            t2 = v * att_vecs[(h * 64 + 16 * k) // 16]
                            acc = t2 if acc is None else acc + t2
                        tbuf[pl.ds(e * 16, 16)] = acc
                    tot = None
                    for k in range(16):
                        col = plsc.load_gather(tbuf, [lanes * 16 + k])
                        tot = col if tot is None else tot + col
                    abuf[h, pl.ds(e0, 16)] = jnp.exp(tot)
                return c2

            lax.fori_loop(0, BE // 16, grp_body, 0)
            dsc = [pltpu.async_copy(
                abuf.at[h, pl.ds(k * CHUNK, CHUNK)],
                den_sh[h].at[dst_v.at[k]], semd, add=True)
                for h in range(H) for k in range(BURST)]
            for h in range(H):
                pltpu.sync_copy(abuf.at[h], alpha_out.at[h, pl.ds(base, BE)])
            for cp in dsc:
                cp.wait()
            return carry

        lax.fori_loop(0, EPT32 // BE, burst_body, 0)
        plsc.subcore_barrier()
        for h in range(H):
            pltpu.sync_copy(den_sh[h].at[pl.ds(sid * ROWS_PT, ROWS_PT)],
                            den_part.at[cid, h, pl.ds(sid * ROWS_PT, ROWS_PT)])

    return alpha_kernel


def _make_agg_kernel(H, D):
    NS = D // 16
    PHASES = NS // 2
    BURST = 16
    BE = BURST * CHUNK
    ZR = 196

    @functools.partial(
        pl.kernel,
        out_type=jax.ShapeDtypeStruct((NS, N_PAD, 16), jnp.float32),
        mesh=_mesh(),
        compiler_params=pltpu.CompilerParams(needs_layout_passes=False, use_tc_tiling_on_sc=False),
        scratch_types=[
            pltpu.VMEM((BE,), jnp.int32),
            pltpu.VMEM((BURST, CHUNK), jnp.int32),
            pltpu.VMEM((BURST, CHUNK), jnp.int32),
            pltpu.VMEM((BE,), jnp.float32),
            pltpu.VMEM((BE, 16), jnp.float32),
            pltpu.VMEM((BE, 16), jnp.float32),
            pltpu.VMEM((ZR, 16), jnp.float32),
            pltpu.VMEM_SHARED((N_PAD, 16), jnp.float32),
            pltpu.SemaphoreType.DMA,
            pltpu.SemaphoreType.DMA,
        ],
    )
    def agg_kernel(xls, srcr, dst2, Pf, agg,
                   src_v, dst_v, idx_v, p_v, rows, msg, zrows,
                   acc, semg, sems):
        cid = lax.axis_index("c")
        sid = lax.axis_index("s")
        zero16 = jnp.zeros((16,), jnp.float32)
        lanes = lax.iota(jnp.int32, 16)
        cvecs = [jnp.full((16,), c, jnp.int32) for c in range(16)]

        def zz(i, c):
            zrows[i, pl.ds(0, 16)] = zero16
            return c

        lax.fori_loop(0, ZR, zz, 0)

        for phase in range(PHASES):
            s = cid * PHASES + phase
            hs = (s * H) // NS
            soff = s * N_PAD
            poff = hs * ET_PAD

            def zr(j, c):
                r0 = sid * ROWS_PT + j * ZR
                pltpu.sync_copy(zrows, acc.at[pl.ds(r0, ZR)])
                return c

            lax.fori_loop(0, ROWS_PT // ZR, zr, 0)
            plsc.subcore_barrier()

            def burst_body(t, c):
                base = sid * EPT16 + t * BE
                row0 = sid * (EPT16 // CHUNK) + t * BURST
                pltpu.sync_copy(srcr.at[pl.ds(base, BE)], src_v)
                pltpu.sync_copy(dst2.at[pl.ds(row0, BURST)], dst_v)
                pltpu.sync_copy(Pf.at[pl.ds(poff + base, BE)], p_v)

                def mk(i, c2):
                    k = i // 8
                    j = i % 8
                    idx_v[k, pl.ds(j * 16, 16)] = (
                        src_v[pl.ds(i * 16, 16)] + soff)
                    return c2

                lax.fori_loop(0, BE // 16, mk, 0)
                cps = [pltpu.async_copy(
                    xls.at[idx_v.at[k]], rows.at[pl.ds(k * CHUNK, CHUNK)], semg)
                    for k in range(BURST)]
                for cp in cps:
                    cp.wait()

                def pe(g, c2):
                    e0 = g * 16
                    ridx = lanes + e0
                    w16 = p_v[pl.ds(e0, 16)]
                    for cc in range(16):
                        col = plsc.load_gather(rows, [ridx, cvecs[cc]])
                        plsc.store_scatter(msg, [ridx, cvecs[cc]], col * w16)
                    return c2

                lax.fori_loop(0, BE // 16, pe, 0)
                sc = [pltpu.async_copy(
                    msg.at[pl.ds(k * CHUNK, CHUNK)], acc.at[dst_v.at[k]],
                    sems, add=True) for k in range(BURST)]
                for cp in sc:
                    cp.wait()
                return c

            lax.fori_loop(0, EPT16 // BE, burst_body, 0)
            plsc.subcore_barrier()

            def fl(j, c):
                r0 = sid * ROWS_PT + j * ZR
                pltpu.sync_copy(acc.at[pl.ds(r0, ZR)], agg.at[s, pl.ds(r0, ZR)])
                return c

            lax.fori_loop(0, ROWS_PT // ZR, fl, 0)
            if phase + 1 < PHASES:
                plsc.subcore_barrier()

    return agg_kernel


def _make_mm_kernel(K, splits):
    Dtot = sum(splits)
    NSxl = splits[0] // 16
    BN = 512
    grid = N_PAD // BN

    def body(x_ref, w_ref, *out_refs):
        y = jnp.dot(x_ref[...], w_ref[...], preferred_element_type=jnp.float32)
        off = 0
        for j, d in enumerate(splits):
            out_refs[j][...] = y[:, off:off + d]
            off += d
        for si in range(NSxl):
            out_refs[-1][si] = y[:, si * 16:(si + 1) * 16]

    outs = ([jax.ShapeDtypeStruct((N_PAD, d), jnp.float32) for d in splits]
            + [jax.ShapeDtypeStruct((NSxl, N_PAD, 16), jnp.float32)])
    out_specs = ([pl.BlockSpec((BN, d), lambda i: (i, 0)) for d in splits]
                 + [pl.BlockSpec((NSxl, BN, 16), lambda i: (0, i, 0))])
    return pl.pallas_call(
        body,
        grid=(grid,),
        in_specs=[pl.BlockSpec((BN, K), lambda i: (i, 0)),
                  pl.BlockSpec((K, Dtot), lambda i: (0, 0))],
        out_specs=out_specs,
        out_shape=outs,
    )


def _make_post_kernel(D, H):
    NS = D // 16
    BN = 256
    grid = N_PAD // BN

    def body(a_ref, d_ref, b_ref, g_ref, bl_ref, sk_ref, skb_ref, o_ref):
        a = a_ref[...]
        dp = d_ref[...]
        den = dp[0] + dp[1] + 1e-16
        parts = []
        for s in range(NS):
            hs = (s * H) // NS
            parts.append(a[s] / den[hs][:, None])
        v = jnp.concatenate(parts, axis=-1) + b_ref[...]
        m = jnp.mean(v, axis=-1, keepdims=True)
        var = jnp.mean((v - m) ** 2, axis=-1, keepdims=True)
        vn = (v - m) * lax.rsqrt(var + 1e-5) * g_ref[...] + bl_ref[...]
        e = jnp.where(vn > 0, vn, jnp.exp(jnp.minimum(vn, 0.0)) - 1.0)
        o_ref[...] = e + sk_ref[...] + skb_ref[...]

    return pl.pallas_call(
        body,
        grid=(grid,),
        in_specs=[pl.BlockSpec((NS, BN, 16), lambda i: (0, i, 0)),
                  pl.BlockSpec((2, H, BN), lambda i: (0, 0, i)),
                  pl.BlockSpec((1, D), lambda i: (0, 0)),
                  pl.BlockSpec((1, D), lambda i: (0, 0)),
                  pl.BlockSpec((1, D), lambda i: (0, 0)),
                  pl.BlockSpec((BN, D), lambda i: (i, 0)),
                  pl.BlockSpec((1, D), lambda i: (0, 0))],
        out_specs=pl.BlockSpec((BN, D), lambda i: (i, 0)),
        out_shape=jax.ShapeDtypeStruct((N_PAD, D), jnp.float32),
    )


def _make_pool_kernel():
    BN = 512
    grid = N_PAD // BN

    def body(h_ref, o_ref):
        i = pl.program_id(0)
        rows = i * BN + lax.broadcasted_iota(jnp.int32, (BN, 64), 0)
        valid = rows < N_NODES
        hb = h_ref[...]
        s = jnp.sum(jnp.where(valid, hb, 0.0), axis=0)
        mx = jnp.max(jnp.where(valid, hb, -jnp.inf), axis=0)

        @pl.when(i == 0)
        def _():
            o_ref[0, :] = s
            o_ref[1, :] = mx

        @pl.when(i > 0)
        def _():
            o_ref[0, :] = o_ref[0, :] + s
            o_ref[1, :] = jnp.maximum(o_ref[1, :], mx)

        @pl.when(i == grid - 1)
        def _():
            o_ref[0, :] = o_ref[0, :] * (1.0 / N_NODES)

    return pl.pallas_call(
        body,
        grid=(grid,),
        in_specs=[pl.BlockSpec((BN, 64), lambda i: (i, 0))],
        out_specs=pl.BlockSpec((2, 64), lambda i: (0, 0)),
        out_shape=jax.ShapeDtypeStruct((2, 64), jnp.float32),
    )


_alpha_2 = _make_alpha_kernel(2, 128)
_alpha_1 = _make_alpha_kernel(1, 64)
_agg_2 = _make_agg_kernel(2, 128)
_agg_1 = _make_agg_kernel(1, 64)
_mm_1 = _make_mm_kernel(16, (128, 128, 128))
_mm_2 = _make_mm_kernel(128, (128, 128))
_mm_3 = _make_mm_kernel(128, (64, 64, 64))
_post_128 = _make_post_kernel(128, 2)
_post_64 = _make_post_kernel(64, 1)
_pool = _make_pool_kernel()


def _gat_layer(xl, xr, xls, src, src2, dst2, att, b, g, bl, sk, skb, H, D):
    NS = D // 16
    alpha_k = _alpha_2 if H == 2 else _alpha_1
    agg_k = _agg_2 if H == 2 else _agg_1
    post = _post_128 if D == 128 else _post_64

    P, den = alpha_k(xl, xr, src2, dst2, att.reshape(-1))
    agg = agg_k(xls.reshape(NS * N_PAD, 16), src, dst2, P.reshape(-1))
    return post(agg, den, b.reshape(1, D), g.reshape(1, D), bl.reshape(1, D),
                sk, skb.reshape(1, D))


def kernel(x, edge_index, Wl1, Wr1, att1, b1, ln1_g, ln1_b, Wsk1, bsk1,
           Wl2, Wr2, att2, b2, ln2_g, ln2_b,
           Wl3, Wr3, att3, b3, ln3_g, ln3_b, Wsk3, bsk3):
    f32 = jnp.float32
    loop = jnp.arange(N_NODES, dtype=jnp.int32)
    pad = jnp.full((ET_PAD - ET,), N_NODES, jnp.int32)
    src = jnp.concatenate([edge_index[0].astype(jnp.int32), loop, pad])
    dst = jnp.concatenate([edge_index[1].astype(jnp.int32), loop, pad])
    src2 = src.reshape(-1, CHUNK)
    dst2 = dst.reshape(-1, CHUNK)

    xp = jnp.zeros((N_PAD, 16), f32).at[:N_NODES, :14].set(x)
    Wt1 = jnp.zeros((16, 384), f32).at[:14].set(
        jnp.concatenate([Wl1.T, Wr1.T, Wsk1.T], axis=1))
    xl1, xr1, xsk1, xl1s = _mm_1(xp, Wt1)
    h1 = _gat_layer(xl1, xr1, xl1s, src, src2, dst2, att1, b1, ln1_g, ln1_b,
                    xsk1, bsk1, H=2, D=128)

    Wt2 = jnp.concatenate([Wl2.T, Wr2.T], axis=1)
    xl2, xr2, xl2s = _mm_2(h1, Wt2)
    h2 = _gat_layer(xl2, xr2, xl2s, src, src2, dst2, att2, b2, ln2_g, ln2_b,
                    h1, jnp.zeros((128,), f32), H=2, D=128)

    Wt3 = jnp.concatenate([Wl3.T, Wr3.T, Wsk3.T], axis=1)
    xl3, xr3, xsk3, xl3s = _mm_3(h2, Wt3)
    h3 = _gat_layer(xl3, xr3, xl3s, src, src2, dst2, att3, b3, ln3_g, ln3_b,
                    xsk3, bsk3, H=1, D=64)

    pooled = _pool(h3)
    return pooled.reshape(1, 128)


# speedup vs baseline: 22.9955x; 1.0962x over previous
import functools

import jax
import jax.numpy as jnp
from jax import lax
from jax.experimental import pallas as pl
from jax.experimental.pallas import tpu as pltpu
from jax.experimental.pallas import tpu_sc as plsc

N_NODES = 50000
N_EDGES = 800000
ET = N_EDGES + N_NODES
N_PAD = 50176
ROWS_PT = N_PAD // 16
CHUNK = 128
ET_PAD = 851968
EPT32 = ET_PAD // 32
EPT16 = ET_PAD // 16


def _mesh():
    return plsc.VectorSubcoreMesh(core_axis_name="c", subcore_axis_name="s",
                                  num_cores=2, num_subcores=16)


def _make_alpha_kernel(H, D):
    NK = D // 16
    BURST = 2
    BE = BURST * CHUNK

    @functools.partial(
        pl.kernel,
        out_type=[jax.ShapeDtypeStruct((H, ET_PAD), jnp.float32),
                  jax.ShapeDtypeStruct((2, H, N_PAD), jnp.float32)],
        mesh=_mesh(),
        compiler_params=pltpu.CompilerParams(needs_layout_passes=False, use_tc_tiling_on_sc=False),
        scratch_types=[
            pltpu.VMEM((BURST, CHUNK), jnp.int32),
            pltpu.VMEM((BURST, CHUNK), jnp.int32),
            pltpu.VMEM((BE, D), jnp.float32),
            pltpu.VMEM((BE, D), jnp.float32),
            pltpu.VMEM((D,), jnp.float32),
            pltpu.VMEM((H, BE), jnp.float32),
            pltpu.VMEM((256,), jnp.float32),
            pltpu.VMEM((ROWS_PT,), jnp.float32),
        ] + [pltpu.VMEM_SHARED((N_PAD,), jnp.float32) for _ in range(H)] + [
            pltpu.SemaphoreType.DMA,
            pltpu.SemaphoreType.DMA,
        ],
    )
    def alpha_kernel(xl, xr, src2, dst2, att, alpha_out, den_part,
                     src_v, dst_v, xj, xi, att_v, abuf, tbuf, zbuf, *rest):
        den_sh = rest[:H]
        semg = rest[H]
        semd = rest[H + 1]
        cid = lax.axis_index("c")
        sid = lax.axis_index("s")
        wid = sid * 2 + cid
        base0 = wid * EPT32
        row0_0 = wid * (EPT32 // CHUNK)
        pltpu.sync_copy(att, att_v)
        att_vecs = [att_v[pl.ds(16 * k, 16)] for k in range(NK)]
        lanes = lax.iota(jnp.int32, 16)
        zero16 = jnp.zeros((16,), jnp.float32)

        def zb(i, c):
            zbuf[pl.ds(i * 16, 16)] = zero16
            return c

        lax.fori_loop(0, ROWS_PT // 16, zb, 0)
        for h in range(H):
            pltpu.sync_copy(zbuf, den_sh[h].at[pl.ds(sid * ROWS_PT, ROWS_PT)])
        plsc.subcore_barrier()

        def burst_body(t, carry):
            base = base0 + t * BE
            row0 = row0_0 + t * BURST
            pltpu.sync_copy(src2.at[pl.ds(row0, BURST)], src_v)
            pltpu.sync_copy(dst2.at[pl.ds(row0, BURST)], dst_v)
            cps = []
            for k in range(BURST):
                cps.append(pltpu.async_copy(
                    xl.at[src_v.at[k]], xj.at[pl.ds(k * CHUNK, CHUNK)], semg))
                cps.append(pltpu.async_copy(
                    xr.at[dst_v.at[k]], xi.at[pl.ds(k * CHUNK, CHUNK)], semg))

            def grp_body(g, c2):
                e0 = g * 16
                for h in range(H):
                    for e in range(16):
                        acc = None
                        for k in range(64 // 16):
                            c = h * 64 + 16 * k
                            v = xj[e0 + e, pl.ds(c, 16)] + xi[e0 + e, pl.ds(c, 16)]
                            v = jnp.maximum(v, 0.2 * v)
                            t2 = v * att_vecs[(h * 64 + 16 * k) // 16]
                            acc = t2 if acc is None else acc + t2
                        tbuf[pl.ds(e * 16, 16)] = acc
                    tot = None
                    for k in range(16):
                        col = plsc.load_gather(tbuf, [lanes * 16 + k])
                        tot = col if tot is None else tot + col
                    abuf[h, pl.ds(e0, 16)] = jnp.exp(tot)
                return c2

            for k in range(BURST):
                cps[2 * k].wait()
                cps[2 * k + 1].wait()
                lax.fori_loop(k * (CHUNK // 16), (k + 1) * (CHUNK // 16),
                              grp_body, 0)
            dsc = [pltpu.async_copy(
                abuf.at[h, pl.ds(k * CHUNK, CHUNK)],
                den_sh[h].at[dst_v.at[k]], semd, add=True)
                for h in range(H) for k in range(BURST)]
            for h in range(H):
                pltpu.sync_copy(abuf.at[h], alpha_out.at[h, pl.ds(base, BE)])
            for cp in dsc:
                cp.wait()
            return carry

        lax.fori_loop(0, EPT32 // BE, burst_body, 0)
        plsc.subcore_barrier()
        for h in range(H):
            pltpu.sync_copy(den_sh[h].at[pl.ds(sid * ROWS_PT, ROWS_PT)],
                            den_part.at[cid, h, pl.ds(sid * ROWS_PT, ROWS_PT)])

    return alpha_kernel


def _make_agg_kernel(H, D):
    NS = D // 16
    PHASES = NS // 2
    BURST = 16
    BE = BURST * CHUNK
    ZR = 196

    @functools.partial(
        pl.kernel,
        out_type=jax.ShapeDtypeStruct((NS, N_PAD, 16), jnp.float32),
        mesh=_mesh(),
        compiler_params=pltpu.CompilerParams(needs_layout_passes=False, use_tc_tiling_on_sc=False),
        scratch_types=[
            pltpu.VMEM((BE,), jnp.int32),
            pltpu.VMEM((BURST, CHUNK), jnp.int32),
            pltpu.VMEM((BURST, CHUNK), jnp.int32),
            pltpu.VMEM((BE,), jnp.float32),
            pltpu.VMEM((BE, 16), jnp.float32),
            pltpu.VMEM((BE, 16), jnp.float32),
            pltpu.VMEM((ZR, 16), jnp.float32),
            pltpu.VMEM_SHARED((N_PAD, 16), jnp.float32),
            pltpu.SemaphoreType.DMA,
            pltpu.SemaphoreType.DMA,
        ],
    )
    def agg_kernel(xls, srcr, dst2, Pf, agg,
                   src_v, dst_v, idx_v, p_v, rows, msg, zrows,
                   acc, semg, sems):
        cid = lax.axis_index("c")
        sid = lax.axis_index("s")
        zero16 = jnp.zeros((16,), jnp.float32)
        lanes = lax.iota(jnp.int32, 16)
        cvecs = [jnp.full((16,), c, jnp.int32) for c in range(16)]

        def zz(i, c):
            zrows[i, pl.ds(0, 16)] = zero16
            return c

        lax.fori_loop(0, ZR, zz, 0)

        def phase_body(ph, c0):
            s = cid * PHASES + ph
            hs = (s * H) // NS
            soff = s * N_PAD
            poff = hs * ET_PAD

            def zr(j, c):
                r0 = sid * ROWS_PT + j * ZR
                pltpu.sync_copy(zrows, acc.at[pl.ds(r0, ZR)])
                return c

            lax.fori_loop(0, ROWS_PT // ZR, zr, 0)
            plsc.subcore_barrier()

            def burst_body(t, c):
                base = sid * EPT16 + t * BE
                row0 = sid * (EPT16 // CHUNK) + t * BURST
                pltpu.sync_copy(srcr.at[pl.ds(base, BE)], src_v)
                pltpu.sync_copy(dst2.at[pl.ds(row0, BURST)], dst_v)
                pltpu.sync_copy(Pf.at[pl.ds(poff + base, BE)], p_v)

                def mk(i, c2):
                    k = i // 8
                    j = i % 8
                    idx_v[k, pl.ds(j * 16, 16)] = (
                        src_v[pl.ds(i * 16, 16)] + soff)
                    return c2

                lax.fori_loop(0, BE // 16, mk, 0)
                cps = [pltpu.async_copy(
                    xls.at[idx_v.at[k]], rows.at[pl.ds(k * CHUNK, CHUNK)], semg)
                    for k in range(BURST)]
                sc = []
                for k in range(BURST):
                    cps[k].wait()

                    def pe(g, c2, k=k):
                        e0 = k * CHUNK + g * 16
                        ridx = lanes + e0
                        w16 = p_v[pl.ds(e0, 16)]
                        for cc in range(16):
                            col = plsc.load_gather(rows, [ridx, cvecs[cc]])
                            plsc.store_scatter(msg, [ridx, cvecs[cc]], col * w16)
                        return c2

                    lax.fori_loop(0, CHUNK // 16, pe, 0)
                    sc.append(pltpu.async_copy(
                        msg.at[pl.ds(k * CHUNK, CHUNK)], acc.at[dst_v.at[k]],
                        sems, add=True))
                for cp in sc:
                    cp.wait()
                return c

            lax.fori_loop(0, EPT16 // BE, burst_body, 0)
            plsc.subcore_barrier()

            def fl(j, c):
                r0 = sid * ROWS_PT + j * ZR
                pltpu.sync_copy(acc.at[pl.ds(r0, ZR)], agg.at[s, pl.ds(r0, ZR)])
                return c

            lax.fori_loop(0, ROWS_PT // ZR, fl, 0)
            plsc.subcore_barrier()
            return c0

        lax.fori_loop(0, PHASES, phase_body, 0)

    return agg_kernel


def _make_mm_kernel(K, splits):
    Dtot = sum(splits)
    NSxl = splits[0] // 16
    BN = 512
    grid = N_PAD // BN

    def body(x_ref, w_ref, *out_refs):
        y = jnp.dot(x_ref[...], w_ref[...], preferred_element_type=jnp.float32)
        off = 0
        for j, d in enumerate(splits):
            out_refs[j][...] = y[:, off:off + d]
            off += d
        for si in range(NSxl):
            out_refs[-1][si] = y[:, si * 16:(si + 1) * 16]

    outs = ([jax.ShapeDtypeStruct((N_PAD, d), jnp.float32) for d in splits]
            + [jax.ShapeDtypeStruct((NSxl, N_PAD, 16), jnp.float32)])
    out_specs = ([pl.BlockSpec((BN, d), lambda i: (i, 0)) for d in splits]
                 + [pl.BlockSpec((NSxl, BN, 16), lambda i: (0, i, 0))])
    return pl.pallas_call(
        body,
        grid=(grid,),
        in_specs=[pl.BlockSpec((BN, K), lambda i: (i, 0)),
                  pl.BlockSpec((K, Dtot), lambda i: (0, 0))],
        out_specs=out_specs,
        out_shape=outs,
    )


def _make_post_kernel(D, H):
    NS = D // 16
    BN = 256
    grid = N_PAD // BN

    def body(a_ref, d_ref, b_ref, g_ref, bl_ref, sk_ref, skb_ref, o_ref):
        a = a_ref[...]
        dp = d_ref[...]
        den = dp[0] + dp[1] + 1e-16
        parts = []
        for s in range(NS):
            hs = (s * H) // NS
            parts.append(a[s] / den[hs][:, None])
        v = jnp.concatenate(parts, axis=-1) + b_ref[...]
        m = jnp.mean(v, axis=-1, keepdims=True)
        var = jnp.mean((v - m) ** 2, axis=-1, keepdims=True)
        vn = (v - m) * lax.rsqrt(var + 1e-5) * g_ref[...] + bl_ref[...]
        e = jnp.where(vn > 0, vn, jnp.exp(jnp.minimum(vn, 0.0)) - 1.0)
        o_ref[...] = e + sk_ref[...] + skb_ref[...]

    return pl.pallas_call(
        body,
        grid=(grid,),
        in_specs=[pl.BlockSpec((NS, BN, 16), lambda i: (0, i, 0)),
                  pl.BlockSpec((2, H, BN), lambda i: (0, 0, i)),
                  pl.BlockSpec((1, D), lambda i: (0, 0)),
                  pl.BlockSpec((1, D), lambda i: (0, 0)),
                  pl.BlockSpec((1, D), lambda i: (0, 0)),
                  pl.BlockSpec((BN, D), lambda i: (i, 0)),
                  pl.BlockSpec((1, D), lambda i: (0, 0))],
        out_specs=pl.BlockSpec((BN, D), lambda i: (i, 0)),
        out_shape=jax.ShapeDtypeStruct((N_PAD, D), jnp.float32),
    )


def _make_pool_kernel():
    BN = 512
    grid = N_PAD // BN

    def body(h_ref, o_ref):
        i = pl.program_id(0)
        rows = i * BN + lax.broadcasted_iota(jnp.int32, (BN, 64), 0)
        valid = rows < N_NODES
        hb = h_ref[...]
        s = jnp.sum(jnp.where(valid, hb, 0.0), axis=0)
        mx = jnp.max(jnp.where(valid, hb, -jnp.inf), axis=0)

        @pl.when(i == 0)
        def _():
            o_ref[0, :] = s
            o_ref[1, :] = mx

        @pl.when(i > 0)
        def _():
            o_ref[0, :] = o_ref[0, :] + s
            o_ref[1, :] = jnp.maximum(o_ref[1, :], mx)

        @pl.when(i == grid - 1)
        def _():
            o_ref[0, :] = o_ref[0, :] * (1.0 / N_NODES)

    return pl.pallas_call(
        body,
        grid=(grid,),
        in_specs=[pl.BlockSpec((BN, 64), lambda i: (i, 0))],
        out_specs=pl.BlockSpec((2, 64), lambda i: (0, 0)),
        out_shape=jax.ShapeDtypeStruct((2, 64), jnp.float32),
    )


_alpha_2 = _make_alpha_kernel(2, 128)
_alpha_1 = _make_alpha_kernel(1, 64)
_agg_2 = _make_agg_kernel(2, 128)
_agg_1 = _make_agg_kernel(1, 64)
_mm_1 = _make_mm_kernel(16, (128, 128, 128))
_mm_2 = _make_mm_kernel(128, (128, 128))
_mm_3 = _make_mm_kernel(128, (64, 64, 64))
_post_128 = _make_post_kernel(128, 2)
_post_64 = _make_post_kernel(64, 1)
_pool = _make_pool_kernel()


def _gat_layer(xl, xr, xls, src, src2, dst2, att, b, g, bl, sk, skb, H, D):
    NS = D // 16
    alpha_k = _alpha_2 if H == 2 else _alpha_1
    agg_k = _agg_2 if H == 2 else _agg_1
    post = _post_128 if D == 128 else _post_64

    P, den = alpha_k(xl, xr, src2, dst2, att.reshape(-1))
    agg = agg_k(xls.reshape(NS * N_PAD, 16), src, dst2, P.reshape(-1))
    return post(agg, den, b.reshape(1, D), g.reshape(1, D), bl.reshape(1, D),
                sk, skb.reshape(1, D))


def kernel(x, edge_index, Wl1, Wr1, att1, b1, ln1_g, ln1_b, Wsk1, bsk1,
           Wl2, Wr2, att2, b2, ln2_g, ln2_b,
           Wl3, Wr3, att3, b3, ln3_g, ln3_b, Wsk3, bsk3):
    f32 = jnp.float32
    loop = jnp.arange(N_NODES, dtype=jnp.int32)
    pad = jnp.full((ET_PAD - ET,), N_NODES, jnp.int32)
    src = jnp.concatenate([edge_index[0].astype(jnp.int32), loop, pad])
    dst = jnp.concatenate([edge_index[1].astype(jnp.int32), loop, pad])
    src2 = src.reshape(-1, CHUNK)
    dst2 = dst.reshape(-1, CHUNK)

    xp = jnp.zeros((N_PAD, 16), f32).at[:N_NODES, :14].set(x)
    Wt1 = jnp.zeros((16, 384), f32).at[:14].set(
        jnp.concatenate([Wl1.T, Wr1.T, Wsk1.T], axis=1))
    xl1, xr1, xsk1, xl1s = _mm_1(xp, Wt1)
    h1 = _gat_layer(xl1, xr1, xl1s, src, src2, dst2, att1, b1, ln1_g, ln1_b,
                    xsk1, bsk1, H=2, D=128)

    Wt2 = jnp.concatenate([Wl2.T, Wr2.T], axis=1)
    xl2, xr2, xl2s = _mm_2(h1, Wt2)
    h2 = _gat_layer(xl2, xr2, xl2s, src, src2, dst2, att2, b2, ln2_g, ln2_b,
                    h1, jnp.zeros((128,), f32), H=2, D=128)

    Wt3 = jnp.concatenate([Wl3.T, Wr3.T, Wsk3.T], axis=1)
    xl3, xr3, xsk3, xl3s = _mm_3(h2, Wt3)
    h3 = _gat_layer(xl3, xr3, xl3s, src, src2, dst2, att3, b3, ln3_g, ln3_b,
                    xsk3, bsk3, H=1, D=64)

    pooled = _pool(h3)
    return pooled.reshape(1, 128)
